# baseline (speedup 1.0000x reference)
import functools

import jax
import jax.numpy as jnp
from jax import lax
from jax.experimental import pallas as pl
from jax.experimental.pallas import tpu as pltpu
from jax.experimental.pallas import tpu_sc as plsc

NW = 32
_INV_SQRT2 = 0.7071067811865476


def _gelu(x):
    return 0.5 * x * (1.0 + lax.erf(x * _INV_SQRT2))

CHUNK = 80


def _pre_body(x_ref, w1_ref, w2_ref, ps_ref, pd_ref):
    x = x_ref[...]
    ps_ref[...] = jnp.dot(x, w1_ref[...], preferred_element_type=jnp.float32)
    pd_ref[...] = jnp.dot(x, w2_ref[...], preferred_element_type=jnp.float32)


def _edge_body(g_ref, el_ref, es_ref, w3a_ref, w3b_ref, bh_ref, wo_ref, bo_ref,
               e_ref):
    q = jnp.dot(el_ref[...], w3a_ref[...], preferred_element_type=jnp.float32)
    q = q + jnp.dot(es_ref[...], w3b_ref[...], preferred_element_type=jnp.float32)
    h = _gelu(g_ref[...] + q + bh_ref[...])
    e_ref[...] = jnp.dot(h, wo_ref[...], preferred_element_type=jnp.float32) + bo_ref[...]


def _node_body(x_ref, p_ref, wna_ref, wnb_ref, bnh_ref, wno_ref, bno_ref,
               out_ref):
    agg = p_ref[0] + p_ref[1]
    h = jnp.dot(x_ref[...], wna_ref[...], preferred_element_type=jnp.float32)
    h = h + jnp.dot(agg, wnb_ref[...], preferred_element_type=jnp.float32)
    h = _gelu(h + bnh_ref[...])
    out_ref[...] = jnp.dot(h, wno_ref[...], preferred_element_type=jnp.float32) + bno_ref[...]


@functools.lru_cache(maxsize=None)
def _make_gather(E, H):
    epw = E // NW
    nchunk = epw // CHUNK
    mesh = plsc.VectorSubcoreMesh(core_axis_name="c", subcore_axis_name="s")

    @functools.partial(
        pl.kernel,
        out_type=jax.ShapeDtypeStruct((E, H), jnp.float32),
        mesh=mesh,
        scratch_types=[
            pltpu.VMEM((epw,), jnp.int32),
            pltpu.VMEM((epw,), jnp.int32),
            pltpu.VMEM((CHUNK, H), jnp.float32),
            pltpu.VMEM((CHUNK, H), jnp.float32),
            pltpu.VMEM((CHUNK, H), jnp.float32),
            pltpu.VMEM((CHUNK, H), jnp.float32),
            pltpu.VMEM((CHUNK, H), jnp.float32),
            pltpu.VMEM((CHUNK, H), jnp.float32),
            pltpu.SemaphoreType.DMA,
            pltpu.SemaphoreType.DMA,
            pltpu.SemaphoreType.DMA,
            pltpu.SemaphoreType.DMA,
            pltpu.SemaphoreType.DMA,
        ],
        compiler_params=pltpu.CompilerParams(use_tc_tiling_on_sc=False),
    )
    def gather_k(ps_hbm, pd_hbm, src_hbm, dst_hbm, out_hbm,
                 idxs_v, idxd_v, bufa0, bufb0, bufs0, bufa1, bufb1, bufs1,
                 si, sg0, sg1, sw0, sw1):
        wid = lax.axis_index("s") * 2 + lax.axis_index("c")
        base = wid * epw
        bufa = (bufa0, bufa1)
        bufb = (bufb0, bufb1)
        bufs = (bufs0, bufs1)
        sgs = (sg0, sg1)
        sws = (sw0, sw1)

        cps = pltpu.async_copy(src_hbm.at[pl.ds(base, epw)], idxs_v, si)
        cpd = pltpu.async_copy(dst_hbm.at[pl.ds(base, epw)], idxd_v, si)
        cps.wait()
        cpd.wait()

        def start_gathers(ci, p):
            isl = idxs_v.at[pl.ds(ci * CHUNK, CHUNK)]
            idl = idxd_v.at[pl.ds(ci * CHUNK, CHUNK)]
            pltpu.async_copy(ps_hbm.at[isl], bufa[p], sgs[p])
            pltpu.async_copy(pd_hbm.at[idl], bufb[p], sgs[p])

        start_gathers(0, 0)
        start_gathers(1, 1)

        def pair(i, carry):
            for p in (0, 1):
                ci = 2 * i + p

                @pl.when(ci < nchunk)
                def _():
                    off = base + ci * CHUNK
                    pltpu.make_async_copy(
                        ps_hbm.at[pl.ds(0, CHUNK)], bufa[p], sgs[p]).wait()
                    pltpu.make_async_copy(
                        pd_hbm.at[pl.ds(0, CHUNK)], bufb[p], sgs[p]).wait()

                    @pl.when(ci >= 2)
                    def _():
                        pltpu.make_async_copy(
                            bufs[p], out_hbm.at[pl.ds(0, CHUNK)], sws[p]).wait()

                    def rows(r8, c2):
                        for rr in range(8):
                            r = r8 * 8 + rr
                            for j in range(H // 16):
                                sl = pl.ds(j * 16, 16)
                                bufs[p][r, sl] = bufa[p][r, sl] + bufb[p][r, sl]
                        return c2

                    lax.fori_loop(0, CHUNK // 8, rows, 0)
                    pltpu.async_copy(bufs[p], out_hbm.at[pl.ds(off, CHUNK)],
                                     sws[p])

                    @pl.when(ci + 2 < nchunk)
                    def _():
                        start_gathers(ci + 2, p)
            return carry

        lax.fori_loop(0, (nchunk + 1) // 2, pair, 0)
        pltpu.make_async_copy(bufs0, out_hbm.at[pl.ds(0, CHUNK)], sw0).wait()
        pltpu.make_async_copy(bufs1, out_hbm.at[pl.ds(0, CHUNK)], sw1).wait()

    return gather_k


@functools.lru_cache(maxsize=None)
def _make_scatter(E, N, L):
    epw = E // NW
    nchunk = epw // CHUNK
    rpt = N // 16
    mesh = plsc.VectorSubcoreMesh(core_axis_name="c", subcore_axis_name="s")

    @functools.partial(
        pl.kernel,
        out_type=jax.ShapeDtypeStruct((2, N, L), jnp.float32),
        mesh=mesh,
        scratch_types=[
            pltpu.VMEM((4, CHUNK), jnp.int32),
            pltpu.VMEM((CHUNK, L), jnp.float32),
            pltpu.VMEM((CHUNK, L), jnp.float32),
            pltpu.VMEM((CHUNK, L), jnp.float32),
            pltpu.VMEM((CHUNK, L), jnp.float32),
            pltpu.VMEM((rpt, L), jnp.float32),
            pltpu.VMEM_SHARED((N, L), jnp.float32),
            pltpu.SemaphoreType.DMA,
            pltpu.SemaphoreType.DMA,
            pltpu.SemaphoreType.DMA,
            pltpu.SemaphoreType.DMA,
            pltpu.SemaphoreType.DMA,
            pltpu.SemaphoreType.DMA,
            pltpu.SemaphoreType.DMA,
            pltpu.SemaphoreType.DMA,
        ],
        compiler_params=pltpu.CompilerParams(use_tc_tiling_on_sc=False),
    )
    def scatter_k(e_hbm, dst_hbm, out_hbm, idx_v,
                  eb0, eb1, eb2, eb3, zbuf, agg_sh,
                  sl0, sl1, sl2, sl3, ss0, ss1, ss2, ss3):
        cid = lax.axis_index("c")
        sid = lax.axis_index("s")
        wid = sid * 2 + cid
        ebuf = (eb0, eb1, eb2, eb3)
        sls = (sl0, sl1, sl2, sl3)
        sss = (ss0, ss1, ss2, ss3)
        base = wid * epw

        def start_loads(ci, p):
            off = base + ci * CHUNK
            pltpu.async_copy(dst_hbm.at[pl.ds(off, CHUNK)], idx_v.at[p], sls[p])
            pltpu.async_copy(e_hbm.at[pl.ds(off, CHUNK)], ebuf[p], sls[p])

        def zrow(i, carry):
            zbuf[i, pl.ds(0, L)] = jnp.zeros((L,), jnp.float32)
            return carry

        lax.fori_loop(0, rpt, zrow, 0)
        for p in range(4):
            start_loads(p, p)
        pltpu.sync_copy(zbuf, agg_sh.at[pl.ds(sid * rpt, rpt)])
        plsc.subcore_barrier()

        def quad(i, carry):
            for p in (0, 1, 2, 3):
                ci = 4 * i + p

                @pl.when(ci < nchunk)
                def _():
                    pltpu.make_async_copy(
                        dst_hbm.at[pl.ds(0, CHUNK)], idx_v.at[p], sls[p]).wait()
                    pltpu.make_async_copy(
                        e_hbm.at[pl.ds(0, CHUNK)], ebuf[p], sls[p]).wait()
                    pltpu.async_copy(ebuf[p], agg_sh.at[idx_v.at[p]], sss[p],
                                     add=True)
                    q = (p + 3) % 4

                    @pl.when(jnp.logical_and(ci >= 1, ci + 3 < nchunk))
                    def _():
                        pltpu.make_async_copy(
                            ebuf[q], agg_sh.at[idx_v.at[q]], sss[q]).wait()
                        start_loads(ci + 3, q)
            return carry

        lax.fori_loop(0, (nchunk + 3) // 4, quad, 0)
        for ci in range(nchunk - 4, nchunk):
            p = ci % 4
            pltpu.make_async_copy(ebuf[p], agg_sh.at[idx_v.at[p]], sss[p]).wait()
        plsc.subcore_barrier()
        pltpu.sync_copy(agg_sh.at[pl.ds(sid * rpt, rpt)],
                        out_hbm.at[cid, pl.ds(sid * rpt, rpt)])

    return scatter_k


def kernel(node_x, edge_index, edge_latent, edge_skip,
           We_h, be_h, We_o, be_o, Wn_h, bn_h, Wn_o, bn_o):
    N, D = node_x.shape
    E = edge_index.shape[1]
    LAT = edge_latent.shape[1]
    SKIP = edge_skip.shape[1]
    HID = We_h.shape[1]

    src = edge_index[0].astype(jnp.int32)
    dst = edge_index[1].astype(jnp.int32)

    W1 = We_h[:D]
    W2 = We_h[D:2 * D]
    W3a = We_h[2 * D:2 * D + LAT]
    W3b = We_h[2 * D + LAT:]

    psrc, pdst = pl.pallas_call(
        _pre_body,
        out_shape=(jax.ShapeDtypeStruct((N, HID), jnp.float32),
                   jax.ShapeDtypeStruct((N, HID), jnp.float32)),
    )(node_x, W1, W2)

    gsum = _make_gather(E, HID)(psrc, pdst, src, dst)

    g2 = gsum.reshape(E // 2, 2 * HID)
    el2 = jnp.concatenate([edge_latent[0::2], edge_latent[1::2]], axis=1)
    es2 = jnp.concatenate([edge_skip[0::2], edge_skip[1::2]], axis=1)
    eye2 = jnp.eye(2, dtype=jnp.float32)
    w3a_bd = jnp.kron(eye2, W3a)
    w3b_bd = jnp.kron(eye2, W3b)
    wo_bd = jnp.kron(eye2, We_o)
    bh2 = jnp.tile(be_h, 2).reshape(1, 2 * HID)
    bo2 = jnp.tile(be_o, 2).reshape(1, 2 * LAT)

    BE = 8000
    grid = E // BE
    e_p8 = pl.pallas_call(
        _edge_body,
        grid=(grid,),
        in_specs=[
            pl.BlockSpec((BE // 2, 2 * HID), lambda i: (i, 0)),
            pl.BlockSpec((BE // 2, 2 * LAT), lambda i: (i, 0)),
            pl.BlockSpec((BE // 2, 2 * SKIP), lambda i: (i, 0)),
            pl.BlockSpec((2 * LAT, 2 * HID), lambda i: (0, 0)),
            pl.BlockSpec((2 * SKIP, 2 * HID), lambda i: (0, 0)),
            pl.BlockSpec((1, 2 * HID), lambda i: (0, 0)),
            pl.BlockSpec((2 * HID, 2 * LAT), lambda i: (0, 0)),
            pl.BlockSpec((1, 2 * LAT), lambda i: (0, 0)),
        ],
        out_specs=pl.BlockSpec((BE // 2, 2 * LAT), lambda i: (i, 0)),
        out_shape=jax.ShapeDtypeStruct((E // 2, 2 * LAT), jnp.float32),
    )(g2, el2, es2, w3a_bd, w3b_bd, bh2, wo_bd, bo2)

    e_dense = e_p8.reshape(E, LAT)

    parts = _make_scatter(E, N, LAT)(e_dense, dst)
    e = e_dense

    x = pl.pallas_call(
        _node_body,
        out_shape=jax.ShapeDtypeStruct((N, D), jnp.float32),
    )(node_x, parts, Wn_h[:D], Wn_h[D:], bn_h.reshape(1, HID),
      Wn_o, bn_o.reshape(1, D))

    return (x, e)


# speedup vs baseline: 2.9361x; 2.9361x over previous
import functools

import jax
import jax.numpy as jnp
from jax import lax
from jax.experimental import pallas as pl
from jax.experimental.pallas import tpu as pltpu
from jax.experimental.pallas import tpu_sc as plsc

NW = 32
_INV_SQRT2 = 0.7071067811865476


def _gelu(x):
    return 0.5 * x * (1.0 + lax.erf(x * _INV_SQRT2))

CHUNK = 80


def _pre_body(x_ref, w1_ref, w2_ref, ps_ref, pd_ref):
    x = x_ref[...]
    ps_ref[...] = jnp.dot(x, w1_ref[...], preferred_element_type=jnp.float32)
    pd_ref[...] = jnp.dot(x, w2_ref[...], preferred_element_type=jnp.float32)


def _edge_body(g_ref, el_ref, es_ref, w3a_ref, w3b_ref, bh_ref, wo_ref, bo_ref,
               e_ref):
    q = jnp.dot(el_ref[...], w3a_ref[...], preferred_element_type=jnp.float32)
    q = q + jnp.dot(es_ref[...], w3b_ref[...], preferred_element_type=jnp.float32)
    h = _gelu(g_ref[...] + q + bh_ref[...])
    e_ref[...] = jnp.dot(h, wo_ref[...], preferred_element_type=jnp.float32) + bo_ref[...]


def _node_body(x_ref, p_ref, wna_ref, wnb_ref, bnh_ref, wno_ref, bno_ref,
               out_ref):
    agg = p_ref[0] + p_ref[1]
    h = jnp.dot(x_ref[...], wna_ref[...], preferred_element_type=jnp.float32)
    h = h + jnp.dot(agg, wnb_ref[...], preferred_element_type=jnp.float32)
    h = _gelu(h + bnh_ref[...])
    out_ref[...] = jnp.dot(h, wno_ref[...], preferred_element_type=jnp.float32) + bno_ref[...]


@functools.lru_cache(maxsize=None)
def _make_gather(E, H):
    epw = E // NW
    nchunk = epw // CHUNK
    mesh = plsc.VectorSubcoreMesh(core_axis_name="c", subcore_axis_name="s")

    @functools.partial(
        pl.kernel,
        out_type=jax.ShapeDtypeStruct((E, H), jnp.float32),
        mesh=mesh,
        scratch_types=[
            pltpu.VMEM((epw,), jnp.int32),
            pltpu.VMEM((epw,), jnp.int32),
            pltpu.VMEM((CHUNK, H), jnp.float32),
            pltpu.VMEM((CHUNK, H), jnp.float32),
            pltpu.VMEM((CHUNK, H), jnp.float32),
            pltpu.VMEM((CHUNK, H), jnp.float32),
            pltpu.VMEM((CHUNK, H), jnp.float32),
            pltpu.VMEM((CHUNK, H), jnp.float32),
            pltpu.SemaphoreType.DMA,
            pltpu.SemaphoreType.DMA,
            pltpu.SemaphoreType.DMA,
            pltpu.SemaphoreType.DMA,
            pltpu.SemaphoreType.DMA,
        ],
        compiler_params=pltpu.CompilerParams(use_tc_tiling_on_sc=False),
    )
    def gather_k(ps_hbm, pd_hbm, src_hbm, dst_hbm, out_hbm,
                 idxs_v, idxd_v, bufa0, bufb0, bufs0, bufa1, bufb1, bufs1,
                 si, sg0, sg1, sw0, sw1):
        wid = lax.axis_index("s") * 2 + lax.axis_index("c")
        base = wid * epw
        bufa = (bufa0, bufa1)
        bufb = (bufb0, bufb1)
        bufs = (bufs0, bufs1)
        sgs = (sg0, sg1)
        sws = (sw0, sw1)

        cps = pltpu.async_copy(src_hbm.at[pl.ds(base, epw)], idxs_v, si)
        cpd = pltpu.async_copy(dst_hbm.at[pl.ds(base, epw)], idxd_v, si)
        cps.wait()
        cpd.wait()

        def start_gathers(ci, p):
            isl = idxs_v.at[pl.ds(ci * CHUNK, CHUNK)]
            idl = idxd_v.at[pl.ds(ci * CHUNK, CHUNK)]
            pltpu.async_copy(ps_hbm.at[isl], bufa[p], sgs[p])
            pltpu.async_copy(pd_hbm.at[idl], bufb[p], sgs[p])

        start_gathers(0, 0)
        start_gathers(1, 1)

        def pair(i, carry):
            for p in (0, 1):
                ci = 2 * i + p

                @pl.when(ci < nchunk)
                def _():
                    off = base + ci * CHUNK
                    pltpu.make_async_copy(
                        ps_hbm.at[pl.ds(0, CHUNK)], bufa[p], sgs[p]).wait()
                    pltpu.make_async_copy(
                        pd_hbm.at[pl.ds(0, CHUNK)], bufb[p], sgs[p]).wait()

                    @pl.when(ci >= 2)
                    def _():
                        pltpu.make_async_copy(
                            bufs[p], out_hbm.at[pl.ds(0, CHUNK)], sws[p]).wait()

                    def rows(r8, c2):
                        for rr in range(8):
                            r = r8 * 8 + rr
                            for j in range(H // 16):
                                sl = pl.ds(j * 16, 16)
                                bufs[p][r, sl] = bufa[p][r, sl] + bufb[p][r, sl]
                        return c2

                    lax.fori_loop(0, CHUNK // 8, rows, 0)
                    pltpu.async_copy(bufs[p], out_hbm.at[pl.ds(off, CHUNK)],
                                     sws[p])

                    @pl.when(ci + 2 < nchunk)
                    def _():
                        start_gathers(ci + 2, p)
            return carry

        lax.fori_loop(0, (nchunk + 1) // 2, pair, 0)
        pltpu.make_async_copy(bufs0, out_hbm.at[pl.ds(0, CHUNK)], sw0).wait()
        pltpu.make_async_copy(bufs1, out_hbm.at[pl.ds(0, CHUNK)], sw1).wait()

    return gather_k


@functools.lru_cache(maxsize=None)
def _make_scatter(E, N, L):
    epw = E // NW
    nchunk = epw // CHUNK
    rpt = N // 16
    mesh = plsc.VectorSubcoreMesh(core_axis_name="c", subcore_axis_name="s")

    @functools.partial(
        pl.kernel,
        out_type=jax.ShapeDtypeStruct((2, N, L), jnp.float32),
        mesh=mesh,
        scratch_types=[
            pltpu.VMEM((4, CHUNK), jnp.int32),
            pltpu.VMEM((CHUNK, L), jnp.float32),
            pltpu.VMEM((CHUNK, L), jnp.float32),
            pltpu.VMEM((CHUNK, L), jnp.float32),
            pltpu.VMEM((CHUNK, L), jnp.float32),
            pltpu.VMEM((rpt, L), jnp.float32),
            pltpu.VMEM_SHARED((N, L), jnp.float32),
            pltpu.SemaphoreType.DMA,
            pltpu.SemaphoreType.DMA,
            pltpu.SemaphoreType.DMA,
            pltpu.SemaphoreType.DMA,
            pltpu.SemaphoreType.DMA,
            pltpu.SemaphoreType.DMA,
            pltpu.SemaphoreType.DMA,
            pltpu.SemaphoreType.DMA,
        ],
        compiler_params=pltpu.CompilerParams(use_tc_tiling_on_sc=False),
    )
    def scatter_k(e_hbm, dst_hbm, out_hbm, idx_v,
                  eb0, eb1, eb2, eb3, zbuf, agg_sh,
                  sl0, sl1, sl2, sl3, ss0, ss1, ss2, ss3):
        cid = lax.axis_index("c")
        sid = lax.axis_index("s")
        wid = sid * 2 + cid
        ebuf = (eb0, eb1, eb2, eb3)
        sls = (sl0, sl1, sl2, sl3)
        sss = (ss0, ss1, ss2, ss3)
        base = wid * epw

        def start_loads(ci, p):
            off = base + ci * CHUNK
            pltpu.async_copy(dst_hbm.at[pl.ds(off, CHUNK)], idx_v.at[p], sls[p])
            pltpu.async_copy(e_hbm.at[pl.ds(off, CHUNK)], ebuf[p], sls[p])

        def zrow(i, carry):
            zbuf[i, pl.ds(0, L)] = jnp.zeros((L,), jnp.float32)
            return carry

        lax.fori_loop(0, rpt, zrow, 0)
        for p in range(4):
            start_loads(p, p)
        pltpu.sync_copy(zbuf, agg_sh.at[pl.ds(sid * rpt, rpt)])
        plsc.subcore_barrier()

        def quad(i, carry):
            for p in (0, 1, 2, 3):
                ci = 4 * i + p

                @pl.when(ci < nchunk)
                def _():
                    pltpu.make_async_copy(
                        dst_hbm.at[pl.ds(0, CHUNK)], idx_v.at[p], sls[p]).wait()
                    pltpu.make_async_copy(
                        e_hbm.at[pl.ds(0, CHUNK)], ebuf[p], sls[p]).wait()
                    pltpu.async_copy(ebuf[p], agg_sh.at[idx_v.at[p]], sss[p],
                                     add=True)
                    q = (p + 3) % 4

                    @pl.when(jnp.logical_and(ci >= 1, ci + 3 < nchunk))
                    def _():
                        pltpu.make_async_copy(
                            ebuf[q], agg_sh.at[idx_v.at[q]], sss[q]).wait()
                        start_loads(ci + 3, q)
            return carry

        lax.fori_loop(0, (nchunk + 3) // 4, quad, 0)
        for ci in range(nchunk - 4, nchunk):
            p = ci % 4
            pltpu.make_async_copy(ebuf[p], agg_sh.at[idx_v.at[p]], sss[p]).wait()
        plsc.subcore_barrier()
        pltpu.sync_copy(agg_sh.at[pl.ds(sid * rpt, rpt)],
                        out_hbm.at[cid, pl.ds(sid * rpt, rpt)])

    return scatter_k


def kernel(node_x, edge_index, edge_latent, edge_skip,
           We_h, be_h, We_o, be_o, Wn_h, bn_h, Wn_o, bn_o):
    N, D = node_x.shape
    E = edge_index.shape[1]
    LAT = edge_latent.shape[1]
    SKIP = edge_skip.shape[1]
    HID = We_h.shape[1]

    src = edge_index[0].astype(jnp.int32)
    dst = edge_index[1].astype(jnp.int32)

    W1 = We_h[:D]
    W2 = We_h[D:2 * D]
    W3a = We_h[2 * D:2 * D + LAT]
    W3b = We_h[2 * D + LAT:]

    psrc, pdst = pl.pallas_call(
        _pre_body,
        out_shape=(jax.ShapeDtypeStruct((N, HID), jnp.float32),
                   jax.ShapeDtypeStruct((N, HID), jnp.float32)),
    )(node_x, W1, W2)

    gsum = _make_gather(E, HID)(psrc, pdst, src, dst)

    g2 = gsum.reshape(E // 2, 2 * HID)
    el2 = edge_latent.astype(jnp.bfloat16).reshape(E // 2, 2 * LAT)
    es2 = edge_skip.astype(jnp.bfloat16).reshape(E // 2, 2 * SKIP)
    eye2 = jnp.eye(2, dtype=jnp.float32)
    w3a_bd = jnp.kron(eye2, W3a).astype(jnp.bfloat16)
    w3b_bd = jnp.kron(eye2, W3b).astype(jnp.bfloat16)
    wo_bd = jnp.kron(eye2, We_o)
    bh2 = jnp.tile(be_h, 2).reshape(1, 2 * HID)
    bo2 = jnp.tile(be_o, 2).reshape(1, 2 * LAT)

    BE = 8000
    grid = E // BE
    e_p8 = pl.pallas_call(
        _edge_body,
        grid=(grid,),
        in_specs=[
            pl.BlockSpec((BE // 2, 2 * HID), lambda i: (i, 0)),
            pl.BlockSpec((BE // 2, 2 * LAT), lambda i: (i, 0)),
            pl.BlockSpec((BE // 2, 2 * SKIP), lambda i: (i, 0)),
            pl.BlockSpec((2 * LAT, 2 * HID), lambda i: (0, 0)),
            pl.BlockSpec((2 * SKIP, 2 * HID), lambda i: (0, 0)),
            pl.BlockSpec((1, 2 * HID), lambda i: (0, 0)),
            pl.BlockSpec((2 * HID, 2 * LAT), lambda i: (0, 0)),
            pl.BlockSpec((1, 2 * LAT), lambda i: (0, 0)),
        ],
        out_specs=pl.BlockSpec((BE // 2, 2 * LAT), lambda i: (i, 0)),
        out_shape=jax.ShapeDtypeStruct((E // 2, 2 * LAT), jnp.float32),
    )(g2, el2, es2, w3a_bd, w3b_bd, bh2, wo_bd, bo2)

    e_dense = e_p8.reshape(E, LAT)

    parts = _make_scatter(E, N, LAT)(e_dense, dst)
    e = lax.optimization_barrier(e_dense)

    x = pl.pallas_call(
        _node_body,
        out_shape=jax.ShapeDtypeStruct((N, D), jnp.float32),
    )(node_x, parts, Wn_h[:D], Wn_h[D:], bn_h.reshape(1, HID),
      Wn_o, bn_o.reshape(1, D))

    return (x, e)
